# baseline (speedup 1.0000x reference)
import functools

import jax
import jax.numpy as jnp
from jax import lax
from jax.experimental import pallas as pl
from jax.experimental.pallas import tpu as pltpu
from jax.experimental.pallas import tpu_sc as plsc

NC = 2
NS = 16
LANES = 16
CH = 80


def _tc1_body(x_ref, wg_ref, att_ref, xw2_ref, a2_ref):
    xw = jnp.dot(x_ref[...], wg_ref[...], preferred_element_type=jnp.float32)
    xw2_ref[...] = xw.reshape(xw2_ref.shape).astype(jnp.bfloat16)
    a2_ref[...] = jnp.dot(xw, att_ref[...], preferred_element_type=jnp.float32)


def _tc1(x, W_gat, att2, rb):
    n, d = x.shape
    h = W_gat.shape[1]
    return pl.pallas_call(
        _tc1_body,
        grid=(n // rb,),
        in_specs=[
            pl.BlockSpec((rb, d), lambda i: (i, 0)),
            pl.BlockSpec((d, h), lambda i: (0, 0)),
            pl.BlockSpec((d, 2), lambda i: (0, 0)),
        ],
        out_specs=[
            pl.BlockSpec((2 * rb, h // 2), lambda i: (i, 0)),
            pl.BlockSpec((rb, 2), lambda i: (i, 0)),
        ],
        out_shape=[
            jax.ShapeDtypeStruct((2 * n, h // 2), jnp.bfloat16),
            jax.ShapeDtypeStruct((n, 2), jnp.float32),
        ],
    )(x, W_gat, att2)


def _make_sc_edge(n, e, hh):
    epc = e // NS
    nchunk = epc // CH
    npad = -(-n // (NS * CH)) * NS * CH
    rpt = npad // NS
    mesh = plsc.VectorSubcoreMesh(core_axis_name="c", subcore_axis_name="s")

    @functools.partial(
        pl.kernel,
        mesh=mesh,
        compiler_params=pltpu.CompilerParams(needs_layout_passes=False,
                                             use_tc_tiling_on_sc=False),
        out_type=[
            jax.ShapeDtypeStruct((NC, npad, hh), jnp.float32),
            jax.ShapeDtypeStruct((NC, npad), jnp.float32),
        ],
        scratch_types=[
            pltpu.VMEM((2 * npad,), jnp.float32),
            pltpu.VMEM((2000,), jnp.int32),
            pltpu.VMEM((2000,), jnp.int32),
            pltpu.VMEM((CH,), jnp.int32),
            pltpu.VMEM((CH,), jnp.int32),
            pltpu.VMEM((CH,), jnp.float32),
            pltpu.VMEM((CH, 64), jnp.int32),
            pltpu.VMEM((CH,), jnp.int32),
            pltpu.VMEM((CH,), jnp.int32),
            pltpu.VMEM((CH,), jnp.float32),
            pltpu.VMEM((CH, 64), jnp.int32),
            pltpu.VMEM((CH, 128), jnp.float32),
            pltpu.VMEM((npad // NS,), jnp.float32),
            pltpu.VMEM_SHARED((npad, 128), jnp.float32),
            pltpu.VMEM_SHARED((npad,), jnp.float32),
            pltpu.SemaphoreType.DMA,
            pltpu.SemaphoreType.DMA,
        ],
    )
    def sc_edge(edge_hbm, a2_hbm, xw2_hbm, numer_hbm, denom_hbm,
                a2_v, srcsb_v, dstsb_v,
                gidx0_v, dst0_v, w0_v, rowsb0_v,
                gidx1_v, dst1_v, w1_v, rowsb1_v,
                rowsf_v, dstage_v, numer_s, denom_s, sem0, sem1):
        c = lax.axis_index("c")
        s = lax.axis_index("s")
        zv = jnp.zeros((LANES,), jnp.float32)
        bufs = ((gidx0_v, dst0_v, w0_v, rowsb0_v, sem0),
                (gidx1_v, dst1_v, w1_v, rowsb1_v, sem1))

        def zrow(i, carry):
            for j in range(128 // LANES):
                rowsf_v[i, pl.ds(j * LANES, LANES)] = zv
            return carry
        lax.fori_loop(0, CH, zrow, 0)

        base = s * rpt
        nfull = rpt // CH

        def zacc(i, carry):
            pltpu.sync_copy(rowsf_v, numer_s.at[pl.ds(base + i * CH, CH)])
            return carry
        lax.fori_loop(0, nfull, zacc, 0)

        def zd(i, carry):
            dstage_v[pl.ds(i * LANES, LANES)] = zv
            return carry
        lax.fori_loop(0, (npad // NS) // LANES, zd, 0)
        pltpu.sync_copy(dstage_v, denom_s.at[pl.ds(base, rpt)])

        pltpu.sync_copy(a2_hbm, a2_v.at[pl.ds(0, 2 * n)])

        def ztail(i, carry):
            a2_v[pl.ds(2 * n + i * LANES, LANES)] = zv
            return carry
        lax.fori_loop(0, (2 * npad - 2 * n) // LANES, ztail, 0)
        plsc.subcore_barrier()

        sbe = 2000
        nck = sbe // CH
        nsb = epc // sbe

        def prep(kk, b):
            gidx_b, dst_b, w_b, rows_b, sem_b = bufs[b]

            def mk(i, carry):
                sv = srcsb_v[pl.ds(kk * CH + i * LANES, LANES)]
                dv = dstsb_v[pl.ds(kk * CH + i * LANES, LANES)]
                gidx_b[pl.ds(i * LANES, LANES)] = sv * 2 + c
                dst_b[pl.ds(i * LANES, LANES)] = dv
                asv = plsc.load_gather(a2_v, [sv * 2])
                adv = plsc.load_gather(a2_v, [dv * 2 + 1])
                al = asv + adv
                al = jnp.where(al >= 0.0, al, al * 0.2)
                w_b[pl.ds(i * LANES, LANES)] = jnp.exp(al)
                return carry
            lax.fori_loop(0, CH // LANES, mk, 0)
            pltpu.async_copy(xw2_hbm.at[gidx_b], rows_b, sem_b)

        def drain(kk, b):
            gidx_b, dst_b, w_b, rows_b, sem_b = bufs[b]
            pltpu.make_async_copy(xw2_hbm.at[gidx_b], rows_b, sem_b).wait()

            def scale(i, carry):
                wv = w_b[pl.ds(i * LANES, LANES)]
                for lane in range(LANES):
                    wi = wv[lane]
                    r = i * LANES + lane
                    for j in range(4):
                        vw = rows_b[r, pl.ds(j * LANES, LANES)]
                        vb = plsc.bitcast(vw, jnp.bfloat16)
                        va, vc = plsc.unpack(vb, format=plsc.PackFormat.INTERLEAVED)
                        rowsf_v[r, pl.ds(j * 32, LANES)] = va * wi
                        rowsf_v[r, pl.ds(j * 32 + LANES, LANES)] = vc * wi
                return carry
            lax.fori_loop(0, CH // LANES, scale, 0)

            pltpu.sync_copy(rowsf_v, numer_s.at[dst_b], add=True)
            pltpu.sync_copy(w_b, denom_s.at[dst_b], add=True)

        def sb_loop(si, carry):
            off = s * epc + si * sbe
            pltpu.sync_copy(edge_hbm.at[pl.ds(off, sbe)], srcsb_v)
            pltpu.sync_copy(edge_hbm.at[pl.ds(e + off, sbe)], dstsb_v)
            prep(0, 0)

            def pair(g, carry2):
                prep(2 * g + 1, 1)
                drain(2 * g, 0)
                prep(2 * g + 2, 0)
                drain(2 * g + 1, 1)
                return carry2
            lax.fori_loop(0, (nck - 1) // 2, pair, 0)
            drain(nck - 1, 0)
            return carry
        lax.fori_loop(0, nsb, sb_loop, 0)

        iota = lax.iota(jnp.int32, LANES)

        def fill(i, carry):
            iv = base + i * LANES + iota
            srcsb_v[pl.ds(i * LANES, LANES)] = jnp.minimum(iv, n - 1)
            dstsb_v[pl.ds(i * LANES, LANES)] = jnp.where(iv < n, iv, npad - 1)
            return carry
        lax.fori_loop(0, rpt // LANES, fill, 0)

        nslc = rpt // CH
        prep(0, 0)
        for k in range(nslc - 1):
            prep(k + 1, (k + 1) % 2)
            drain(k, k % 2)
        drain(nslc - 1, (nslc - 1) % 2)
        plsc.subcore_barrier()

        def dump(i, carry):
            r0 = base + i * CH
            pltpu.sync_copy(numer_s.at[pl.ds(r0, CH)], rowsf_v)
            pltpu.sync_copy(rowsf_v, numer_hbm.at[c, pl.ds(r0, CH)])
            return carry
        lax.fori_loop(0, nfull, dump, 0)

        pltpu.sync_copy(denom_s.at[pl.ds(base, rpt)], dstage_v)
        pltpu.sync_copy(dstage_v, denom_hbm.at[c, pl.ds(base, rpt)])

    return sc_edge


def _tc2_body(n0_ref, n1_ref, den_ref, b_ref, wih_ref, out_ref):
    rb, h = out_ref.shape
    numer = jnp.concatenate(
        [n0_ref[...].reshape(rb, h // 2), n1_ref[...].reshape(rb, h // 2)],
        axis=1)
    og = jnp.tanh(numer / den_ref[...] + b_ref[...])
    wih = jnp.concatenate([wih_ref[0:h, :], wih_ref[2 * h:4 * h, :]], axis=0)
    g = lax.dot_general(og, wih, (((1,), (1,)), ((), ())),
                        preferred_element_type=jnp.float32)
    i_g = g[:, 0:h]
    g_g = g[:, h:2 * h]
    o_g = g[:, 2 * h:3 * h]
    c1 = jax.nn.sigmoid(i_g) * jnp.tanh(g_g)
    out_ref[...] = jax.nn.sigmoid(o_g) * jnp.tanh(c1)


def _tc2(numer, den, bias, W_ih, n, rb):
    hh = numer.shape[2]
    h = 2 * hh
    return pl.pallas_call(
        _tc2_body,
        grid=(n // rb,),
        in_specs=[
            pl.BlockSpec((1, rb, hh), lambda i: (0, i, 0)),
            pl.BlockSpec((1, rb, hh), lambda i: (1, i, 0)),
            pl.BlockSpec((rb, 1), lambda i: (i, 0)),
            pl.BlockSpec((1, h), lambda i: (0, 0)),
            pl.BlockSpec((4 * h, h), lambda i: (0, 0)),
        ],
        out_specs=pl.BlockSpec((rb, h), lambda i: (i, 0)),
        out_shape=jax.ShapeDtypeStruct((n, h), jnp.float32),
    )(numer, numer, den, bias, W_ih)


def kernel(x, edge_index, W_gat, att_src, att_dst, bias_gat, W_ih, W_hh):
    n, d = x.shape
    h = W_gat.shape[1]
    e = edge_index.shape[1]
    hh = h // 2
    rb = 2000

    perm = jnp.arange(h).reshape(h // 32, 2, 16).transpose(0, 2, 1).reshape(h)
    wg_p = W_gat[:, perm]
    att2 = jnp.stack([att_src, att_dst], axis=1)[perm, :]
    xw2, a2 = _tc1(x, wg_p, att2, rb)
    a2f = a2.reshape(2 * n)

    xw2i = lax.bitcast_convert_type(xw2.reshape(2 * n, hh // 2, 2), jnp.int32)
    numer, denom = _make_sc_edge(n, e, hh)(edge_index.reshape(2 * e), a2f, xw2i)

    return _tc2(numer, denom[0, :n].reshape(n, 1),
                bias_gat.reshape(1, h), W_ih, n, rb)


# speedup vs baseline: 2.1614x; 2.1614x over previous
import functools

import jax
import jax.numpy as jnp
from jax import lax
from jax.experimental import pallas as pl
from jax.experimental.pallas import tpu as pltpu
from jax.experimental.pallas import tpu_sc as plsc

NC = 2
NS = 16
LANES = 16
CH = 80


def _tc1_body(x_ref, wg_ref, att_ref, xw2_ref, a2_ref):
    xw = jnp.dot(x_ref[...], wg_ref[...], preferred_element_type=jnp.float32)
    xw2_ref[...] = xw.reshape(xw2_ref.shape)
    a2_ref[...] = jnp.dot(xw, att_ref[...], preferred_element_type=jnp.float32)


def _tc1(x, W_gat, att2, rb):
    n, d = x.shape
    h = W_gat.shape[1]
    return pl.pallas_call(
        _tc1_body,
        grid=(n // rb,),
        in_specs=[
            pl.BlockSpec((rb, d), lambda i: (i, 0)),
            pl.BlockSpec((d, h), lambda i: (0, 0)),
            pl.BlockSpec((d, 2), lambda i: (0, 0)),
        ],
        out_specs=[
            pl.BlockSpec((2 * rb, h // 2), lambda i: (i, 0)),
            pl.BlockSpec((rb, 2), lambda i: (i, 0)),
        ],
        out_shape=[
            jax.ShapeDtypeStruct((2 * n, h // 2), jnp.float32),
            jax.ShapeDtypeStruct((n, 2), jnp.float32),
        ],
    )(x, W_gat, att2)


def _make_sc_edge(n, e, hh):
    epc = e // NS
    nchunk = epc // CH
    npad = -(-n // (NS * CH)) * NS * CH
    rpt = npad // NS
    mesh = plsc.VectorSubcoreMesh(core_axis_name="c", subcore_axis_name="s")

    @functools.partial(
        pl.kernel,
        mesh=mesh,
        compiler_params=pltpu.CompilerParams(needs_layout_passes=False),
        out_type=[
            jax.ShapeDtypeStruct((NC, npad, hh), jnp.float32),
            jax.ShapeDtypeStruct((NC, npad), jnp.float32),
        ],
        scratch_types=[
            pltpu.VMEM((2 * npad,), jnp.float32),
            pltpu.VMEM((2000,), jnp.int32),
            pltpu.VMEM((2000,), jnp.int32),
            pltpu.VMEM((CH,), jnp.int32),
            pltpu.VMEM((CH,), jnp.int32),
            pltpu.VMEM((CH,), jnp.float32),
            pltpu.VMEM((CH, 128), jnp.float32),
            pltpu.VMEM((CH,), jnp.int32),
            pltpu.VMEM((CH,), jnp.int32),
            pltpu.VMEM((CH,), jnp.float32),
            pltpu.VMEM((CH, 128), jnp.float32),
            pltpu.VMEM((npad // NS,), jnp.float32),
            pltpu.VMEM_SHARED((npad, 128), jnp.float32),
            pltpu.VMEM_SHARED((npad,), jnp.float32),
            pltpu.SemaphoreType.DMA,
            pltpu.SemaphoreType.DMA,
        ],
    )
    def sc_edge(edge_hbm, a2_hbm, xw2_hbm, numer_hbm, denom_hbm,
                a2_v, srcsb_v, dstsb_v,
                gidx0_v, dst0_v, w0_v, rowsb0_v,
                gidx1_v, dst1_v, w1_v, rowsb1_v,
                dstage_v, numer_s, denom_s, sem0, sem1):
        c = lax.axis_index("c")
        s = lax.axis_index("s")
        zv = jnp.zeros((LANES,), jnp.float32)
        bufs = ((gidx0_v, dst0_v, w0_v, rowsb0_v, sem0),
                (gidx1_v, dst1_v, w1_v, rowsb1_v, sem1))

        def zrow(i, carry):
            for j in range(128 // LANES):
                rowsb0_v[i, pl.ds(j * LANES, LANES)] = zv
            return carry
        lax.fori_loop(0, CH, zrow, 0)

        base = s * rpt
        nfull = rpt // CH

        def zacc(i, carry):
            pltpu.sync_copy(rowsb0_v, numer_s.at[pl.ds(base + i * CH, CH)])
            return carry
        lax.fori_loop(0, nfull, zacc, 0)

        def zd(i, carry):
            dstage_v[pl.ds(i * LANES, LANES)] = zv
            return carry
        lax.fori_loop(0, (npad // NS) // LANES, zd, 0)
        pltpu.sync_copy(dstage_v, denom_s.at[pl.ds(base, rpt)])

        pltpu.sync_copy(a2_hbm, a2_v.at[pl.ds(0, 2 * n)])

        def ztail(i, carry):
            a2_v[pl.ds(2 * n + i * LANES, LANES)] = zv
            return carry
        lax.fori_loop(0, (2 * npad - 2 * n) // LANES, ztail, 0)
        plsc.subcore_barrier()

        sbe = 2000
        nck = sbe // CH
        nsb = epc // sbe

        def prep(kk, b):
            gidx_b, dst_b, w_b, rows_b, sem_b = bufs[b]

            def mk(i, carry):
                sv = srcsb_v[pl.ds(kk * CH + i * LANES, LANES)]
                dv = dstsb_v[pl.ds(kk * CH + i * LANES, LANES)]
                gidx_b[pl.ds(i * LANES, LANES)] = sv * 2 + c
                dst_b[pl.ds(i * LANES, LANES)] = dv
                asv = plsc.load_gather(a2_v, [sv * 2])
                adv = plsc.load_gather(a2_v, [dv * 2 + 1])
                al = asv + adv
                al = jnp.where(al >= 0.0, al, al * 0.2)
                w_b[pl.ds(i * LANES, LANES)] = jnp.exp(al)
                return carry
            lax.fori_loop(0, CH // LANES, mk, 0)
            pltpu.async_copy(xw2_hbm.at[gidx_b], rows_b, sem_b)

        def drain(kk, b):
            gidx_b, dst_b, w_b, rows_b, sem_b = bufs[b]
            pltpu.make_async_copy(xw2_hbm.at[gidx_b], rows_b, sem_b).wait()

            def scale(i, carry):
                wv = w_b[pl.ds(i * LANES, LANES)]
                for lane in range(LANES):
                    wi = wv[lane]
                    r = i * LANES + lane
                    for j in range(128 // LANES):
                        sl = pl.ds(j * LANES, LANES)
                        rows_b[r, sl] = rows_b[r, sl] * wi
                return carry
            lax.fori_loop(0, CH // LANES, scale, 0)

            pltpu.sync_copy(rows_b, numer_s.at[dst_b], add=True)
            pltpu.sync_copy(w_b, denom_s.at[dst_b], add=True)

        def sb_loop(si, carry):
            off = s * epc + si * sbe
            pltpu.sync_copy(edge_hbm.at[pl.ds(off, sbe)], srcsb_v)
            pltpu.sync_copy(edge_hbm.at[pl.ds(e + off, sbe)], dstsb_v)
            prep(0, 0)

            def pair(g, carry2):
                prep(2 * g + 1, 1)
                drain(2 * g, 0)
                prep(2 * g + 2, 0)
                drain(2 * g + 1, 1)
                return carry2
            lax.fori_loop(0, (nck - 1) // 2, pair, 0)
            drain(nck - 1, 0)
            return carry
        lax.fori_loop(0, nsb, sb_loop, 0)

        iota = lax.iota(jnp.int32, LANES)

        def fill(i, carry):
            iv = base + i * LANES + iota
            srcsb_v[pl.ds(i * LANES, LANES)] = jnp.minimum(iv, n - 1)
            dstsb_v[pl.ds(i * LANES, LANES)] = jnp.where(iv < n, iv, npad - 1)
            return carry
        lax.fori_loop(0, rpt // LANES, fill, 0)

        nslc = rpt // CH
        prep(0, 0)
        for k in range(nslc - 1):
            prep(k + 1, (k + 1) % 2)
            drain(k, k % 2)
        drain(nslc - 1, (nslc - 1) % 2)
        plsc.subcore_barrier()

        def dump(i, carry):
            r0 = base + i * CH
            pltpu.sync_copy(numer_s.at[pl.ds(r0, CH)], rowsb0_v)
            pltpu.sync_copy(rowsb0_v, numer_hbm.at[c, pl.ds(r0, CH)])
            return carry
        lax.fori_loop(0, nfull, dump, 0)

        pltpu.sync_copy(denom_s.at[pl.ds(base, rpt)], dstage_v)
        pltpu.sync_copy(dstage_v, denom_hbm.at[c, pl.ds(base, rpt)])

    return sc_edge


def _tc2_body(n0_ref, n1_ref, den_ref, b_ref, wih_ref, out_ref):
    rb, h = out_ref.shape
    numer = jnp.concatenate(
        [n0_ref[...].reshape(rb, h // 2), n1_ref[...].reshape(rb, h // 2)],
        axis=1)
    og = jnp.tanh(numer / den_ref[...] + b_ref[...])
    wih = jnp.concatenate([wih_ref[0:h, :], wih_ref[2 * h:4 * h, :]], axis=0)
    g = lax.dot_general(og, wih, (((1,), (1,)), ((), ())),
                        preferred_element_type=jnp.float32)
    i_g = g[:, 0:h]
    g_g = g[:, h:2 * h]
    o_g = g[:, 2 * h:3 * h]
    c1 = jax.nn.sigmoid(i_g) * jnp.tanh(g_g)
    out_ref[...] = jax.nn.sigmoid(o_g) * jnp.tanh(c1)


def _tc2(numer, den, bias, W_ih, n, rb):
    hh = numer.shape[2]
    h = 2 * hh
    return pl.pallas_call(
        _tc2_body,
        grid=(n // rb,),
        in_specs=[
            pl.BlockSpec((1, rb, hh), lambda i: (0, i, 0)),
            pl.BlockSpec((1, rb, hh), lambda i: (1, i, 0)),
            pl.BlockSpec((rb, 1), lambda i: (i, 0)),
            pl.BlockSpec((1, h), lambda i: (0, 0)),
            pl.BlockSpec((4 * h, h), lambda i: (0, 0)),
        ],
        out_specs=pl.BlockSpec((rb, h), lambda i: (i, 0)),
        out_shape=jax.ShapeDtypeStruct((n, h), jnp.float32),
    )(numer, numer, den, bias, W_ih)


def kernel(x, edge_index, W_gat, att_src, att_dst, bias_gat, W_ih, W_hh):
    n, d = x.shape
    h = W_gat.shape[1]
    e = edge_index.shape[1]
    hh = h // 2
    rb = 2000

    att2 = jnp.stack([att_src, att_dst], axis=1)
    xw2, a2 = _tc1(x, W_gat, att2, rb)
    a2f = a2.reshape(2 * n)

    numer, denom = _make_sc_edge(n, e, hh)(edge_index.reshape(2 * e), a2f, xw2)

    return _tc2(numer, denom[0, :n].reshape(n, 1),
                bias_gat.reshape(1, h), W_ih, n, rb)


# speedup vs baseline: 2.3140x; 1.0706x over previous
import functools

import jax
import jax.numpy as jnp
from jax import lax
from jax.experimental import pallas as pl
from jax.experimental.pallas import tpu as pltpu
from jax.experimental.pallas import tpu_sc as plsc

NC = 2
NS = 16
LANES = 16
CH = 80


def _tc1_body(x_ref, wg_ref, att_ref, xw2_ref, a2_ref):
    xw = jnp.dot(x_ref[...], wg_ref[...], preferred_element_type=jnp.float32)
    xw2_ref[...] = xw.reshape(xw2_ref.shape)
    a2_ref[...] = jnp.dot(xw, att_ref[...], preferred_element_type=jnp.float32)


def _tc1(x, W_gat, att2, rb):
    n, d = x.shape
    h = W_gat.shape[1]
    return pl.pallas_call(
        _tc1_body,
        grid=(n // rb,),
        in_specs=[
            pl.BlockSpec((rb, d), lambda i: (i, 0)),
            pl.BlockSpec((d, h), lambda i: (0, 0)),
            pl.BlockSpec((d, 2), lambda i: (0, 0)),
        ],
        out_specs=[
            pl.BlockSpec((2 * rb, h // 2), lambda i: (i, 0)),
            pl.BlockSpec((rb, 2), lambda i: (i, 0)),
        ],
        out_shape=[
            jax.ShapeDtypeStruct((2 * n, h // 2), jnp.float32),
            jax.ShapeDtypeStruct((n, 2), jnp.float32),
        ],
    )(x, W_gat, att2)


def _make_sc_edge(n, e, hh):
    epc = e // NS
    nchunk = epc // CH
    npad = -(-n // (NS * CH)) * NS * CH
    rpt = npad // NS
    mesh = plsc.VectorSubcoreMesh(core_axis_name="c", subcore_axis_name="s")

    @functools.partial(
        pl.kernel,
        mesh=mesh,
        compiler_params=pltpu.CompilerParams(needs_layout_passes=False),
        out_type=[
            jax.ShapeDtypeStruct((NC, npad, hh), jnp.float32),
            jax.ShapeDtypeStruct((NC, npad), jnp.float32),
        ],
        scratch_types=[
            pltpu.VMEM((2 * n,), jnp.float32),
            pltpu.VMEM((2000,), jnp.int32),
            pltpu.VMEM((2000,), jnp.int32),
            pltpu.VMEM((CH,), jnp.int32),
            pltpu.VMEM((CH,), jnp.int32),
            pltpu.VMEM((CH,), jnp.float32),
            pltpu.VMEM((CH, 128), jnp.float32),
            pltpu.VMEM((CH,), jnp.int32),
            pltpu.VMEM((CH,), jnp.int32),
            pltpu.VMEM((CH,), jnp.float32),
            pltpu.VMEM((CH, 128), jnp.float32),
            pltpu.VMEM((npad // NS,), jnp.float32),
            pltpu.VMEM_SHARED((npad, 128), jnp.float32),
            pltpu.VMEM_SHARED((npad,), jnp.float32),
            pltpu.SemaphoreType.DMA,
            pltpu.SemaphoreType.DMA,
        ],
    )
    def sc_edge(edge_hbm, a2_hbm, xw2_hbm, numer_hbm, denom_hbm,
                a2_v, srcsb_v, dstsb_v,
                gidx0_v, dst0_v, w0_v, rowsb0_v,
                gidx1_v, dst1_v, w1_v, rowsb1_v,
                dstage_v, numer_s, denom_s, sem0, sem1):
        c = lax.axis_index("c")
        s = lax.axis_index("s")
        zv = jnp.zeros((LANES,), jnp.float32)
        bufs = ((gidx0_v, dst0_v, w0_v, rowsb0_v, sem0),
                (gidx1_v, dst1_v, w1_v, rowsb1_v, sem1))

        def zrow(i, carry):
            for j in range(128 // LANES):
                rowsb0_v[i, pl.ds(j * LANES, LANES)] = zv
            return carry
        lax.fori_loop(0, CH, zrow, 0)

        base = s * rpt
        nfull = rpt // CH

        def zacc(i, carry):
            pltpu.sync_copy(rowsb0_v, numer_s.at[pl.ds(base + i * CH, CH)])
            return carry
        lax.fori_loop(0, nfull, zacc, 0)

        def zd(i, carry):
            dstage_v[pl.ds(i * LANES, LANES)] = zv
            return carry
        lax.fori_loop(0, (npad // NS) // LANES, zd, 0)
        pltpu.sync_copy(dstage_v, denom_s.at[pl.ds(base, rpt)])

        pltpu.sync_copy(a2_hbm, a2_v)
        plsc.subcore_barrier()

        sbe = 2000
        nck = sbe // CH
        nsb = epc // sbe

        def prep(kk, b):
            gidx_b, dst_b, w_b, rows_b, sem_b = bufs[b]

            def mk(i, carry):
                sv = srcsb_v[pl.ds(kk * CH + i * LANES, LANES)]
                dv = dstsb_v[pl.ds(kk * CH + i * LANES, LANES)]
                gidx_b[pl.ds(i * LANES, LANES)] = sv * 2 + c
                dst_b[pl.ds(i * LANES, LANES)] = dv
                asv = plsc.load_gather(a2_v, [sv * 2])
                adv = plsc.load_gather(a2_v, [dv * 2 + 1])
                al = asv + adv
                al = jnp.where(al >= 0.0, al, al * 0.2)
                w_b[pl.ds(i * LANES, LANES)] = jnp.exp(al)
                return carry
            lax.fori_loop(0, CH // LANES, mk, 0)
            pltpu.async_copy(xw2_hbm.at[gidx_b], rows_b, sem_b)

        def drain(kk, b):
            gidx_b, dst_b, w_b, rows_b, sem_b = bufs[b]
            pltpu.make_async_copy(xw2_hbm.at[gidx_b], rows_b, sem_b).wait()

            def scale(i, carry):
                wv = w_b[pl.ds(i * LANES, LANES)]
                for lane in range(LANES):
                    wi = wv[lane]
                    r = i * LANES + lane
                    for j in range(128 // LANES):
                        sl = pl.ds(j * LANES, LANES)
                        rows_b[r, sl] = rows_b[r, sl] * wi
                return carry
            lax.fori_loop(0, CH // LANES, scale, 0)

            pltpu.sync_copy(rows_b, numer_s.at[dst_b], add=True)
            pltpu.sync_copy(w_b, denom_s.at[dst_b], add=True)

        def sb_loop(si, carry):
            off = s * epc + si * sbe
            pltpu.sync_copy(edge_hbm.at[pl.ds(off, sbe)], srcsb_v)
            pltpu.sync_copy(edge_hbm.at[pl.ds(e + off, sbe)], dstsb_v)
            prep(0, 0)

            def pair(g, carry2):
                prep(2 * g + 1, 1)
                drain(2 * g, 0)
                prep(2 * g + 2, 0)
                drain(2 * g + 1, 1)
                return carry2
            lax.fori_loop(0, (nck - 1) // 2, pair, 0)
            drain(nck - 1, 0)
            return carry
        lax.fori_loop(0, nsb, sb_loop, 0)
        plsc.subcore_barrier()

        def dump(i, carry):
            r0 = base + i * CH
            pltpu.sync_copy(numer_s.at[pl.ds(r0, CH)], rowsb0_v)
            pltpu.sync_copy(rowsb0_v, numer_hbm.at[c, pl.ds(r0, CH)])
            return carry
        lax.fori_loop(0, nfull, dump, 0)

        pltpu.sync_copy(denom_s.at[pl.ds(base, rpt)], dstage_v)
        pltpu.sync_copy(dstage_v, denom_hbm.at[c, pl.ds(base, rpt)])

    return sc_edge


def _tc2_body(n0_ref, n1_ref, den_ref, a2_ref, xw2_ref, b_ref, wih_ref, out_ref):
    rb, h = out_ref.shape
    a2b = a2_ref[...]
    al = a2b[:, 0:1] + a2b[:, 1:2]
    al = jnp.where(al >= 0.0, al, 0.2 * al)
    ws = jnp.exp(al)
    xwb = xw2_ref[...].reshape(rb, h)
    numer = jnp.concatenate(
        [n0_ref[...].reshape(rb, h // 2), n1_ref[...].reshape(rb, h // 2)],
        axis=1) + ws * xwb
    den = den_ref[...] + ws
    og = jnp.tanh(numer / den + b_ref[...])
    wih = jnp.concatenate([wih_ref[0:h, :], wih_ref[2 * h:4 * h, :]], axis=0)
    g = lax.dot_general(og, wih, (((1,), (1,)), ((), ())),
                        preferred_element_type=jnp.float32)
    i_g = g[:, 0:h]
    g_g = g[:, h:2 * h]
    o_g = g[:, 2 * h:3 * h]
    c1 = jax.nn.sigmoid(i_g) * jnp.tanh(g_g)
    out_ref[...] = jax.nn.sigmoid(o_g) * jnp.tanh(c1)


def _tc2(numer, den, a2, xw2, bias, W_ih, rb):
    hh = numer.shape[2]
    h = 2 * hh
    n = a2.shape[0]
    return pl.pallas_call(
        _tc2_body,
        grid=(n // rb,),
        in_specs=[
            pl.BlockSpec((1, rb, hh), lambda i: (0, i, 0)),
            pl.BlockSpec((1, rb, hh), lambda i: (1, i, 0)),
            pl.BlockSpec((rb, 1), lambda i: (i, 0)),
            pl.BlockSpec((rb, 2), lambda i: (i, 0)),
            pl.BlockSpec((2 * rb, hh), lambda i: (i, 0)),
            pl.BlockSpec((1, h), lambda i: (0, 0)),
            pl.BlockSpec((4 * h, h), lambda i: (0, 0)),
        ],
        out_specs=pl.BlockSpec((rb, h), lambda i: (i, 0)),
        out_shape=jax.ShapeDtypeStruct((n, h), jnp.float32),
    )(numer, numer, den, a2, xw2, bias, W_ih)


def kernel(x, edge_index, W_gat, att_src, att_dst, bias_gat, W_ih, W_hh):
    n, d = x.shape
    h = W_gat.shape[1]
    e = edge_index.shape[1]
    hh = h // 2
    rb = 2000

    att2 = jnp.stack([att_src, att_dst], axis=1)
    xw2, a2 = _tc1(x, W_gat, att2, rb)
    a2f = a2.reshape(2 * n)

    numer, denom = _make_sc_edge(n, e, hh)(edge_index.reshape(2 * e), a2f, xw2)

    return _tc2(numer, denom[0, :n].reshape(n, 1), a2, xw2,
                bias_gat.reshape(1, h), W_ih, rb)


# speedup vs baseline: 2.3369x; 1.0099x over previous
import functools

import jax
import jax.numpy as jnp
from jax import lax
from jax.experimental import pallas as pl
from jax.experimental.pallas import tpu as pltpu
from jax.experimental.pallas import tpu_sc as plsc

NC = 2
NS = 16
LANES = 16
CH = 80


def _tc1_body(x_ref, wg_ref, att_ref, xw2_ref, a2_ref):
    xw = jnp.dot(x_ref[...], wg_ref[...], preferred_element_type=jnp.float32)
    xw2_ref[...] = xw.reshape(xw2_ref.shape)
    a2_ref[...] = jnp.dot(xw, att_ref[...], preferred_element_type=jnp.float32)


def _tc1(x, W_gat, att2, rb):
    n, d = x.shape
    h = W_gat.shape[1]
    return pl.pallas_call(
        _tc1_body,
        grid=(n // rb,),
        in_specs=[
            pl.BlockSpec((rb, d), lambda i: (i, 0)),
            pl.BlockSpec((d, h), lambda i: (0, 0)),
            pl.BlockSpec((d, 2), lambda i: (0, 0)),
        ],
        out_specs=[
            pl.BlockSpec((2 * rb, h // 2), lambda i: (i, 0)),
            pl.BlockSpec((rb, 2), lambda i: (i, 0)),
        ],
        out_shape=[
            jax.ShapeDtypeStruct((2 * n, h // 2), jnp.float32),
            jax.ShapeDtypeStruct((n, 2), jnp.float32),
        ],
    )(x, W_gat, att2)


def _make_sc_edge(n, e, hh):
    epc = e // NS
    nchunk = epc // CH
    npad = -(-n // (NS * CH)) * NS * CH
    rpt = npad // NS
    mesh = plsc.VectorSubcoreMesh(core_axis_name="c", subcore_axis_name="s")

    @functools.partial(
        pl.kernel,
        mesh=mesh,
        compiler_params=pltpu.CompilerParams(needs_layout_passes=False),
        out_type=[
            jax.ShapeDtypeStruct((NC, npad, hh), jnp.float32),
            jax.ShapeDtypeStruct((NC, npad), jnp.float32),
        ],
        scratch_types=[
            pltpu.VMEM((2 * n,), jnp.float32),
            pltpu.VMEM((2000,), jnp.int32),
            pltpu.VMEM((2000,), jnp.int32),
            pltpu.VMEM((CH,), jnp.int32),
            pltpu.VMEM((CH,), jnp.int32),
            pltpu.VMEM((CH,), jnp.float32),
            pltpu.VMEM((CH, 128), jnp.float32),
            pltpu.VMEM((CH,), jnp.int32),
            pltpu.VMEM((CH,), jnp.int32),
            pltpu.VMEM((CH,), jnp.float32),
            pltpu.VMEM((CH, 128), jnp.float32),
            pltpu.VMEM((npad // NS,), jnp.float32),
            pltpu.VMEM_SHARED((npad, 128), jnp.float32),
            pltpu.VMEM_SHARED((npad,), jnp.float32),
            pltpu.SemaphoreType.DMA,
            pltpu.SemaphoreType.DMA,
            pltpu.SemaphoreType.DMA,
            pltpu.SemaphoreType.DMA,
        ],
    )
    def sc_edge(edge_hbm, a2_hbm, xw2_hbm, numer_hbm, denom_hbm,
                a2_v, srcsb_v, dstsb_v,
                gidx0_v, dst0_v, w0_v, rowsb0_v,
                gidx1_v, dst1_v, w1_v, rowsb1_v,
                dstage_v, numer_s, denom_s, sem0, sem1, semsc0, semsc1):
        c = lax.axis_index("c")
        s = lax.axis_index("s")
        zv = jnp.zeros((LANES,), jnp.float32)
        bufs = ((gidx0_v, dst0_v, w0_v, rowsb0_v, sem0, semsc0),
                (gidx1_v, dst1_v, w1_v, rowsb1_v, sem1, semsc1))

        ziv = jnp.zeros((LANES,), jnp.int32)

        def zrow(i, carry):
            for j in range(128 // LANES):
                rowsb0_v[i, pl.ds(j * LANES, LANES)] = zv
                rowsb1_v[i, pl.ds(j * LANES, LANES)] = zv
            return carry
        lax.fori_loop(0, CH, zrow, 0)

        def zidx(i, carry):
            sl = pl.ds(i * LANES, LANES)
            dst0_v[sl] = ziv
            dst1_v[sl] = ziv
            w0_v[sl] = zv
            w1_v[sl] = zv
            return carry
        lax.fori_loop(0, CH // LANES, zidx, 0)

        base = s * rpt
        nfull = rpt // CH

        def zacc(i, carry):
            pltpu.sync_copy(rowsb0_v, numer_s.at[pl.ds(base + i * CH, CH)])
            return carry
        lax.fori_loop(0, nfull, zacc, 0)

        def zd(i, carry):
            dstage_v[pl.ds(i * LANES, LANES)] = zv
            return carry
        lax.fori_loop(0, (npad // NS) // LANES, zd, 0)
        pltpu.sync_copy(dstage_v, denom_s.at[pl.ds(base, rpt)])

        pltpu.sync_copy(a2_hbm, a2_v)
        plsc.subcore_barrier()

        sbe = 2000
        nck = sbe // CH
        nsb = epc // sbe

        def prep(kk, b):
            gidx_b, dst_b, w_b, rows_b, sem_b, semsc_b = bufs[b]
            pltpu.make_async_copy(rows_b, numer_s.at[dst_b], semsc_b).wait()
            pltpu.make_async_copy(w_b, denom_s.at[dst_b], semsc_b).wait()

            def mk(i, carry):
                sv = srcsb_v[pl.ds(kk * CH + i * LANES, LANES)]
                dv = dstsb_v[pl.ds(kk * CH + i * LANES, LANES)]
                gidx_b[pl.ds(i * LANES, LANES)] = sv * 2 + c
                dst_b[pl.ds(i * LANES, LANES)] = dv
                asv = plsc.load_gather(a2_v, [sv * 2])
                adv = plsc.load_gather(a2_v, [dv * 2 + 1])
                al = asv + adv
                al = jnp.where(al >= 0.0, al, al * 0.2)
                w_b[pl.ds(i * LANES, LANES)] = jnp.exp(al)
                return carry
            lax.fori_loop(0, CH // LANES, mk, 0)
            pltpu.async_copy(xw2_hbm.at[gidx_b], rows_b, sem_b)

        def drain(kk, b):
            gidx_b, dst_b, w_b, rows_b, sem_b, semsc_b = bufs[b]
            pltpu.make_async_copy(xw2_hbm.at[gidx_b], rows_b, sem_b).wait()

            def scale(i, carry):
                wv = w_b[pl.ds(i * LANES, LANES)]
                for lane in range(LANES):
                    wi = wv[lane]
                    r = i * LANES + lane
                    for j in range(128 // LANES):
                        sl = pl.ds(j * LANES, LANES)
                        rows_b[r, sl] = rows_b[r, sl] * wi
                return carry
            lax.fori_loop(0, CH // LANES, scale, 0)

            pltpu.async_copy(rows_b, numer_s.at[dst_b], semsc_b, add=True)
            pltpu.async_copy(w_b, denom_s.at[dst_b], semsc_b, add=True)

        for _, dst_b, w_b, rows_b, _, semsc_b in bufs:
            pltpu.async_copy(rows_b, numer_s.at[dst_b], semsc_b, add=True)
            pltpu.async_copy(w_b, denom_s.at[dst_b], semsc_b, add=True)

        def sb_loop(si, carry):
            off = s * epc + si * sbe
            pltpu.sync_copy(edge_hbm.at[pl.ds(off, sbe)], srcsb_v)
            pltpu.sync_copy(edge_hbm.at[pl.ds(e + off, sbe)], dstsb_v)
            prep(0, 0)

            def pair(g, carry2):
                prep(2 * g + 1, 1)
                drain(2 * g, 0)
                prep(2 * g + 2, 0)
                drain(2 * g + 1, 1)
                return carry2
            lax.fori_loop(0, (nck - 1) // 2, pair, 0)
            drain(nck - 1, 0)
            return carry
        lax.fori_loop(0, nsb, sb_loop, 0)
        for _, dst_b, w_b, rows_b, _, semsc_b in bufs:
            pltpu.make_async_copy(rows_b, numer_s.at[dst_b], semsc_b).wait()
            pltpu.make_async_copy(w_b, denom_s.at[dst_b], semsc_b).wait()
        plsc.subcore_barrier()

        def dump(i, carry):
            r0 = base + i * CH
            pltpu.sync_copy(numer_s.at[pl.ds(r0, CH)], rowsb0_v)
            pltpu.sync_copy(rowsb0_v, numer_hbm.at[c, pl.ds(r0, CH)])
            return carry
        lax.fori_loop(0, nfull, dump, 0)

        pltpu.sync_copy(denom_s.at[pl.ds(base, rpt)], dstage_v)
        pltpu.sync_copy(dstage_v, denom_hbm.at[c, pl.ds(base, rpt)])

    return sc_edge


def _tc2_body(n0_ref, n1_ref, den_ref, a2_ref, xw2_ref, b_ref, wih_ref, out_ref):
    rb, h = out_ref.shape
    a2b = a2_ref[...]
    al = a2b[:, 0:1] + a2b[:, 1:2]
    al = jnp.where(al >= 0.0, al, 0.2 * al)
    ws = jnp.exp(al)
    xwb = xw2_ref[...].reshape(rb, h)
    numer = jnp.concatenate(
        [n0_ref[...].reshape(rb, h // 2), n1_ref[...].reshape(rb, h // 2)],
        axis=1) + ws * xwb
    den = den_ref[...] + ws
    og = jnp.tanh(numer / den + b_ref[...])
    wih = jnp.concatenate([wih_ref[0:h, :], wih_ref[2 * h:4 * h, :]], axis=0)
    g = lax.dot_general(og, wih, (((1,), (1,)), ((), ())),
                        preferred_element_type=jnp.float32)
    i_g = g[:, 0:h]
    g_g = g[:, h:2 * h]
    o_g = g[:, 2 * h:3 * h]
    c1 = jax.nn.sigmoid(i_g) * jnp.tanh(g_g)
    out_ref[...] = jax.nn.sigmoid(o_g) * jnp.tanh(c1)


def _tc2(numer, den, a2, xw2, bias, W_ih, rb):
    hh = numer.shape[2]
    h = 2 * hh
    n = a2.shape[0]
    return pl.pallas_call(
        _tc2_body,
        grid=(n // rb,),
        in_specs=[
            pl.BlockSpec((1, rb, hh), lambda i: (0, i, 0)),
            pl.BlockSpec((1, rb, hh), lambda i: (1, i, 0)),
            pl.BlockSpec((rb, 1), lambda i: (i, 0)),
            pl.BlockSpec((rb, 2), lambda i: (i, 0)),
            pl.BlockSpec((2 * rb, hh), lambda i: (i, 0)),
            pl.BlockSpec((1, h), lambda i: (0, 0)),
            pl.BlockSpec((4 * h, h), lambda i: (0, 0)),
        ],
        out_specs=pl.BlockSpec((rb, h), lambda i: (i, 0)),
        out_shape=jax.ShapeDtypeStruct((n, h), jnp.float32),
    )(numer, numer, den, a2, xw2, bias, W_ih)


def kernel(x, edge_index, W_gat, att_src, att_dst, bias_gat, W_ih, W_hh):
    n, d = x.shape
    h = W_gat.shape[1]
    e = edge_index.shape[1]
    hh = h // 2
    rb = 2000

    att2 = jnp.stack([att_src, att_dst], axis=1)
    xw2, a2 = _tc1(x, W_gat, att2, rb)
    a2f = a2.reshape(2 * n)

    numer, denom = _make_sc_edge(n, e, hh)(edge_index.reshape(2 * e), a2f, xw2)

    return _tc2(numer, denom[0, :n].reshape(n, 1), a2, xw2,
                bias_gat.reshape(1, h), W_ih, rb)
